# baseline (speedup 1.0000x reference)
import functools

import jax
import jax.numpy as jnp
from jax import lax
from jax.experimental import pallas as pl
from jax.experimental.pallas import tpu as pltpu
from jax.experimental.pallas import tpu_sc as plsc

VOCAB = 1000000
EMBED_DIM = 32
BATCH = 16384
HIST = 50

NC, NS = 2, 16
NW = NC * NS
XPW = BATCH // NW
XR = 16
CHUNK = XR * HIST
NCHUNK = XPW // XR

_mesh = plsc.VectorSubcoreMesh(core_axis_name="c", subcore_axis_name="s")


@functools.partial(
    pl.kernel,
    out_type=jax.ShapeDtypeStruct((BATCH, HIST, EMBED_DIM), jnp.float32),
    mesh=_mesh,
    scratch_types=[
        pltpu.VMEM((XPW, HIST), jnp.int32),
        pltpu.VMEM((2, CHUNK, EMBED_DIM), jnp.float32),
        pltpu.SemaphoreType.DMA((2,)),
        pltpu.SemaphoreType.DMA((2,)),
    ],
    compiler_params=pltpu.CompilerParams(use_tc_tiling_on_sc=False),
)
def _gather(x_hbm, table_hbm, out_hbm, idx_v, rows_v, gsem, wsem):
    wid = lax.axis_index("s") * NC + lax.axis_index("c")
    xbase = wid * XPW

    pltpu.sync_copy(x_hbm.at[pl.ds(xbase, XPW)], idx_v)

    def fire(j, b):
        for k in range(XR):
            pltpu.async_copy(
                table_hbm.at[idx_v.at[j * XR + k]],
                rows_v.at[b, pl.ds(k * HIST, HIST)],
                gsem.at[b],
            )

    def gwait(j, b):
        for k in range(XR):
            pltpu.make_async_copy(
                table_hbm.at[idx_v.at[j * XR + k]],
                rows_v.at[b, pl.ds(k * HIST, HIST)],
                gsem.at[b],
            ).wait()

    def wb(j, b):
        x0 = xbase + j * XR
        for k in range(XR):
            pltpu.async_copy(
                rows_v.at[b, pl.ds(k * HIST, HIST)],
                out_hbm.at[x0 + k],
                wsem.at[b],
            )
        for k in range(XR):
            pltpu.make_async_copy(
                rows_v.at[b, pl.ds(k * HIST, HIST)],
                out_hbm.at[x0 + k],
                wsem.at[b],
            ).wait()

    fire(0, 0)
    fire(1, 1)

    def step(i, carry):
        for b in range(2):
            j = 2 * i + b
            gwait(j, b)
            wb(j, b)
            fire(j + 2, b)
        return carry

    lax.fori_loop(0, NCHUNK // 2 - 1, step, 0)
    for b in range(2):
        j = NCHUNK - 2 + b
        gwait(j, b)
        wb(j, b)


def kernel(x, table):
    return _gather(x.astype(jnp.int32), table)


# speedup vs baseline: 1.0004x; 1.0004x over previous
import functools

import jax
import jax.numpy as jnp
from jax import lax
from jax.experimental import pallas as pl
from jax.experimental.pallas import tpu as pltpu
from jax.experimental.pallas import tpu_sc as plsc

VOCAB = 1000000
EMBED_DIM = 32
BATCH = 16384
HIST = 50

NC, NS = 2, 16
NW = NC * NS
XPW = BATCH // NW
XR = 16
CHUNK = XR * HIST
NCHUNK = XPW // XR

_mesh = plsc.VectorSubcoreMesh(core_axis_name="c", subcore_axis_name="s")


@functools.partial(
    pl.kernel,
    out_type=jax.ShapeDtypeStruct((BATCH, HIST, EMBED_DIM), jnp.float32),
    mesh=_mesh,
    scratch_types=[
        pltpu.VMEM((XPW, HIST), jnp.int32),
        pltpu.VMEM((2, CHUNK, EMBED_DIM), jnp.float32),
        pltpu.SemaphoreType.DMA((2,)),
        pltpu.SemaphoreType.DMA((2,)),
    ],
    compiler_params=pltpu.CompilerParams(
        use_tc_tiling_on_sc=False,
        skip_device_barrier=True,
    ),
)
def _gather(x_hbm, table_hbm, out_hbm, idx_v, rows_v, gsem, wsem):
    wid = lax.axis_index("s") * NC + lax.axis_index("c")
    xbase = wid * XPW

    pltpu.sync_copy(x_hbm.at[pl.ds(xbase, XPW)], idx_v)

    def fire(j, b):
        for k in range(XR):
            pltpu.async_copy(
                table_hbm.at[idx_v.at[j * XR + k]],
                rows_v.at[b, pl.ds(k * HIST, HIST)],
                gsem.at[b],
            )

    def gwait(j, b):
        for k in range(XR):
            pltpu.make_async_copy(
                table_hbm.at[idx_v.at[j * XR + k]],
                rows_v.at[b, pl.ds(k * HIST, HIST)],
                gsem.at[b],
            ).wait()

    def wb(j, b):
        x0 = xbase + j * XR
        for k in range(XR):
            pltpu.async_copy(
                rows_v.at[b, pl.ds(k * HIST, HIST)],
                out_hbm.at[x0 + k],
                wsem.at[b],
            )
        for k in range(XR):
            pltpu.make_async_copy(
                rows_v.at[b, pl.ds(k * HIST, HIST)],
                out_hbm.at[x0 + k],
                wsem.at[b],
            ).wait()

    fire(0, 0)
    fire(1, 1)

    def step(i, carry):
        for b in range(2):
            j = 2 * i + b
            gwait(j, b)
            wb(j, b)
            fire(j + 2, b)
        return carry

    lax.fori_loop(0, NCHUNK // 2 - 1, step, 0)
    for b in range(2):
        j = NCHUNK - 2 + b
        gwait(j, b)
        wb(j, b)


def kernel(x, table):
    return _gather(x.astype(jnp.int32), table)


# speedup vs baseline: 1.0057x; 1.0053x over previous
import functools

import jax
import jax.numpy as jnp
from jax import lax
from jax.experimental import pallas as pl
from jax.experimental.pallas import tpu as pltpu
from jax.experimental.pallas import tpu_sc as plsc

VOCAB = 1000000
EMBED_DIM = 32
BATCH = 16384
HIST = 50

B_TOTAL = BATCH * HIST
NC, NS = 2, 16
NW = NC * NS
BPW = B_TOTAL // NW
CHUNK = 1600
NCHUNK = BPW // CHUNK
XR = CHUNK // HIST
XPW = BATCH // NW

_mesh = plsc.VectorSubcoreMesh(core_axis_name="c", subcore_axis_name="s")


@functools.partial(
    pl.kernel,
    out_type=jax.ShapeDtypeStruct((BATCH, HIST, EMBED_DIM), jnp.float32),
    mesh=_mesh,
    scratch_types=[
        pltpu.VMEM((BPW,), jnp.int32),
        pltpu.VMEM((2, CHUNK, EMBED_DIM), jnp.float32),
        pltpu.SemaphoreType.DMA((2,)),
        pltpu.SemaphoreType.DMA((2,)),
    ],
    compiler_params=pltpu.CompilerParams(use_tc_tiling_on_sc=False),
)
def _gather(idx_hbm, table_hbm, out_hbm, idx_v, rows_v, gsem, wsem):
    wid = lax.axis_index("s") * NC + lax.axis_index("c")
    base = wid * BPW
    xbase = wid * XPW

    pltpu.sync_copy(idx_hbm.at[pl.ds(base, BPW)], idx_v)

    def fire(j, b):
        pltpu.async_copy(
            table_hbm.at[idx_v.at[pl.ds(j * CHUNK, CHUNK)]],
            rows_v.at[b],
            gsem.at[b],
        )

    def gwait(j, b):
        pltpu.make_async_copy(
            table_hbm.at[idx_v.at[pl.ds(j * CHUNK, CHUNK)]],
            rows_v.at[b],
            gsem.at[b],
        ).wait()

    def wb(j, b):
        x0 = xbase + j * XR
        for k in range(XR):
            pltpu.async_copy(
                rows_v.at[b, pl.ds(k * HIST, HIST)],
                out_hbm.at[x0 + k],
                wsem.at[b],
            )
        for k in range(XR):
            pltpu.make_async_copy(
                rows_v.at[b, pl.ds(k * HIST, HIST)],
                out_hbm.at[x0 + k],
                wsem.at[b],
            ).wait()

    fire(0, 0)
    fire(1, 1)

    def step(i, carry):
        for b in range(2):
            j = 2 * i + b
            gwait(j, b)
            wb(j, b)
            fire(j + 2, b)
        return carry

    lax.fori_loop(0, NCHUNK // 2 - 1, step, 0)
    for b in range(2):
        j = NCHUNK - 2 + b
        gwait(j, b)
        wb(j, b)


def kernel(x, table):
    flat = x.reshape(B_TOTAL).astype(jnp.int32)
    return _gather(flat, table)
